# baseline (speedup 1.0000x reference)
import functools

import jax
import jax.numpy as jnp
from jax import lax
from jax.experimental import pallas as pl
from jax.experimental.pallas import tpu as pltpu
from jax.experimental.pallas import tpu_sc as plsc

WEIGHT_DECAY = 0.001
NCORES = 2
NSUB = 16
NW = NCORES * NSUB
CH = 128
FIRE = 8
PB = 2048


def _sc_gather(table, idx2, name):
    m2, ch = idx2.shape
    m = m2 * ch
    d = table.shape[1]
    dt = table.dtype
    per_w = m2 // NW
    groups = per_w // FIRE
    buf_rows = FIRE * ch

    mesh = plsc.VectorSubcoreMesh(core_axis_name="c", subcore_axis_name="s",
                                  num_cores=NCORES, num_subcores=NSUB)

    @functools.partial(
        pl.kernel,
        mesh=mesh,
        name=name,
        compiler_params=pltpu.CompilerParams(use_tc_tiling_on_sc=False),
        out_type=jax.ShapeDtypeStruct((m, d), dt),
        scratch_types=[
            pltpu.VMEM((per_w, ch), jnp.int32),
            pltpu.VMEM((buf_rows, d), dt),
            pltpu.SemaphoreType.DMA,
        ],
    )
    def gather_kernel(tbl_hbm, idx_hbm, g_hbm, idx_v, rows_v, sem):
        wid = lax.axis_index("s") * NCORES + lax.axis_index("c")
        pltpu.sync_copy(idx_hbm.at[pl.ds(wid * per_w, per_w)], idx_v)
        for grp in range(groups):
            for j in range(FIRE):
                pltpu.make_async_copy(
                    tbl_hbm.at[idx_v.at[grp * FIRE + j]],
                    rows_v.at[pl.ds(j * ch, ch)], sem).start()
            for j in range(FIRE):
                pltpu.make_async_copy(
                    tbl_hbm.at[idx_v.at[grp * FIRE + j]],
                    rows_v.at[pl.ds(j * ch, ch)], sem).wait()
            base = (wid * per_w + grp * FIRE) * ch
            pltpu.sync_copy(rows_v, g_hbm.at[pl.ds(base, buf_rows)])

    return gather_kernel(table, idx2)


def _tc_compute(g2, edge_w2, b, w_win, ns, d, name):
    bw = b * w_win
    pb2 = PB // 2
    nj = b // PB
    m2 = g2.shape[0]
    ids_blk = (2 * m2 - b) // PB
    un_blk = bw // PB
    vn_blk = (bw + bw * ns) // PB
    bw_blk = bw // PB

    def body(*refs):
        ids_r, out_r_ = refs[0], refs[1]
        un_rs = refs[2:2 + ns]
        vn_rs = refs[2 + ns:2 + 2 * ns]
        w_r = refs[2 + 2 * ns]
        acc_r = refs[3 + 2 * ns]

        w2 = w_r[0, :]
        u2 = ids_r[...].astype(jnp.float32)
        v2 = out_r_[...].astype(jnp.float32)

        def logsig(x):
            return jnp.log(jax.nn.sigmoid(jnp.clip(x, -6.0, 6.0)))

        lane = jax.lax.broadcasted_iota(jnp.int32, (2 * d, 2), 0)
        col = jax.lax.broadcasted_iota(jnp.int32, (2 * d, 2), 1)
        sel = ((lane // d) == col).astype(jnp.float32)

        def hdots(q):
            return jax.lax.dot_general(
                q, sel, (((1,), (0,)), ((), ())),
                preferred_element_type=jnp.float32)

        def dotsum(q):
            return jnp.sum(logsig(hdots(q)))

        def ndotsum(q):
            return jnp.sum(logsig(-hdots(q)))

        acc = dotsum(u2 * v2 * w2)
        sq = jnp.sum(u2 * u2) + jnp.sum(v2 * v2)
        s = jnp.float32(0.0)
        for nrs, base in ((un_rs, v2), (vn_rs, u2)):
            for k in range(ns):
                nk = nrs[k][...].astype(jnp.float32)
                s += ndotsum(nk * base * w2)
                sq += jnp.sum(nk * nk)

        contrib = -(acc + 0.5 * s - 0.5 * WEIGHT_DECAY * sq) / b

        @pl.when((pl.program_id(0) == 0) & (pl.program_id(1) == 0))
        def _():
            acc_r[0, 0] = 0.0

        acc_r[0, 0] += contrib

    def blk(off_blocks):
        return pl.BlockSpec(
            (pb2, 2 * d), lambda wi, j, o=off_blocks: (o + wi * nj + j, 0))

    in_specs = ([pl.BlockSpec((pb2, 2 * d), lambda wi, j: (ids_blk + j, 0)),
                 blk(0)]
                + [blk(un_blk + k * bw_blk) for k in range(ns)]
                + [blk(vn_blk + k * bw_blk) for k in range(ns)]
                + [pl.BlockSpec((1, 2 * d), lambda wi, j: (0, 0))])
    operands = [g2] * (2 + 2 * ns) + [edge_w2]
    return pl.pallas_call(
        body,
        grid=(w_win, nj),
        in_specs=in_specs,
        out_specs=pl.BlockSpec(memory_space=pltpu.SMEM),
        out_shape=jax.ShapeDtypeStruct((1, 1), jnp.float32),
        name=name,
    )(*operands)


def kernel(input_labels, out_labels, in_embed, out_embed, edge_w,
           u_noise, v_noise, num_sampled):
    del num_sampled
    b, w1 = out_labels.shape
    w_win = w1 - 1
    d = in_embed.shape[1]
    ns = u_noise.shape[1]

    ids = input_labels[:, 1].astype(jnp.int32)
    out_t = out_labels[:, 1:].reshape(-1).astype(jnp.int32)
    idx = jnp.concatenate([out_t,
                           u_noise.T.reshape(-1).astype(jnp.int32),
                           v_noise.T.reshape(-1).astype(jnp.int32),
                           ids])
    m = idx.shape[0]
    idx2 = idx.reshape(m // CH, CH)

    edge_w2 = jnp.concatenate([edge_w, edge_w]).reshape(1, 2 * d)

    g_in = _sc_gather(in_embed.astype(jnp.bfloat16), idx2, "gather_in")
    g_out = _sc_gather(out_embed.astype(jnp.bfloat16), idx2, "gather_out")
    res_in = _tc_compute(g_in.reshape(m // 2, 2 * d), edge_w2,
                         b, w_win, ns, d, "loss_in")
    res_out = _tc_compute(g_out.reshape(m // 2, 2 * d), edge_w2,
                          b, w_win, ns, d, "loss_out")
    return res_in[0, 0] + res_out[0, 0]


# speedup vs baseline: 1.4756x; 1.4756x over previous
import functools

import jax
import jax.numpy as jnp
from jax import lax
from jax.experimental import pallas as pl
from jax.experimental.pallas import tpu as pltpu
from jax.experimental.pallas import tpu_sc as plsc

WEIGHT_DECAY = 0.001
NCORES = 2
NSUB = 16
NW = NCORES * NSUB
CH = 128
FIRE = 8
PB = 2048


def _sc_gather(table, idx2, name):
    m2, ch = idx2.shape
    m = m2 * ch
    d = table.shape[1]
    per_w = m2 // NW
    groups = per_w // FIRE
    buf_rows = FIRE * ch

    mesh = plsc.VectorSubcoreMesh(core_axis_name="c", subcore_axis_name="s",
                                  num_cores=NCORES, num_subcores=NSUB)

    @functools.partial(
        pl.kernel,
        mesh=mesh,
        name=name,
        compiler_params=pltpu.CompilerParams(use_tc_tiling_on_sc=False),
        out_type=jax.ShapeDtypeStruct((m, d), jnp.float32),
        scratch_types=[
            pltpu.VMEM((per_w, ch), jnp.int32),
            pltpu.VMEM((buf_rows, d), jnp.float32),
            pltpu.SemaphoreType.DMA,
        ],
    )
    def gather_kernel(tbl_hbm, idx_hbm, g_hbm, idx_v, rows_v, sem):
        wid = lax.axis_index("s") * NCORES + lax.axis_index("c")
        pltpu.sync_copy(idx_hbm.at[pl.ds(wid * per_w, per_w)], idx_v)
        for grp in range(groups):
            for j in range(FIRE):
                pltpu.make_async_copy(
                    tbl_hbm.at[idx_v.at[grp * FIRE + j]],
                    rows_v.at[pl.ds(j * ch, ch)], sem).start()
            for j in range(FIRE):
                pltpu.make_async_copy(
                    tbl_hbm.at[idx_v.at[grp * FIRE + j]],
                    rows_v.at[pl.ds(j * ch, ch)], sem).wait()
            base = (wid * per_w + grp * FIRE) * ch
            pltpu.sync_copy(rows_v, g_hbm.at[pl.ds(base, buf_rows)])

    return gather_kernel(table, idx2)


def _tc_compute(g2, edge_w2, b, w_win, ns, d, name):
    bw = b * w_win
    pb2 = PB // 2
    nj = b // PB
    m2 = g2.shape[0]
    ids_blk = (2 * m2 - b) // PB
    un_blk = bw // PB
    vn_blk = (bw + bw * ns) // PB
    bw_blk = bw // PB

    def body(*refs):
        ids_r, out_r_ = refs[0], refs[1]
        un_rs = refs[2:2 + ns]
        vn_rs = refs[2 + ns:2 + 2 * ns]
        w_r = refs[2 + 2 * ns]
        acc_r = refs[3 + 2 * ns]

        w2 = w_r[0, :]
        u2 = ids_r[...]
        v2 = out_r_[...]

        def logsig(x):
            return jnp.log(jax.nn.sigmoid(jnp.clip(x, -6.0, 6.0)))

        lane = jax.lax.broadcasted_iota(jnp.int32, (2 * d, 2), 0)
        col = jax.lax.broadcasted_iota(jnp.int32, (2 * d, 2), 1)
        sel = ((lane // d) == col).astype(jnp.float32)

        def hdots(q):
            return jax.lax.dot_general(
                q, sel, (((1,), (0,)), ((), ())),
                preferred_element_type=jnp.float32)

        def dotsum(q):
            return jnp.sum(logsig(hdots(q)))

        def ndotsum(q):
            return jnp.sum(logsig(-hdots(q)))

        acc = dotsum(u2 * v2 * w2)
        sq = jnp.sum(u2 * u2) + jnp.sum(v2 * v2)
        s = jnp.float32(0.0)
        for nrs, base in ((un_rs, v2), (vn_rs, u2)):
            for k in range(ns):
                nk = nrs[k][...]
                s += ndotsum(nk * base * w2)
                sq += jnp.sum(nk * nk)

        contrib = -(acc + 0.5 * s - 0.5 * WEIGHT_DECAY * sq) / b

        @pl.when((pl.program_id(0) == 0) & (pl.program_id(1) == 0))
        def _():
            acc_r[0, 0] = 0.0

        acc_r[0, 0] += contrib

    def blk(off_blocks):
        return pl.BlockSpec(
            (pb2, 2 * d), lambda wi, j, o=off_blocks: (o + wi * nj + j, 0))

    in_specs = ([pl.BlockSpec((pb2, 2 * d), lambda wi, j: (ids_blk + j, 0)),
                 blk(0)]
                + [blk(un_blk + k * bw_blk) for k in range(ns)]
                + [blk(vn_blk + k * bw_blk) for k in range(ns)]
                + [pl.BlockSpec((1, 2 * d), lambda wi, j: (0, 0))])
    operands = [g2] * (2 + 2 * ns) + [edge_w2]
    return pl.pallas_call(
        body,
        grid=(w_win, nj),
        in_specs=in_specs,
        out_specs=pl.BlockSpec(memory_space=pltpu.SMEM),
        out_shape=jax.ShapeDtypeStruct((1, 1), jnp.float32),
        name=name,
    )(*operands)


def kernel(input_labels, out_labels, in_embed, out_embed, edge_w,
           u_noise, v_noise, num_sampled):
    del num_sampled
    b, w1 = out_labels.shape
    w_win = w1 - 1
    d = in_embed.shape[1]
    ns = u_noise.shape[1]

    ids = input_labels[:, 1].astype(jnp.int32)
    out_t = out_labels[:, 1:].reshape(-1).astype(jnp.int32)
    idx = jnp.concatenate([out_t,
                           u_noise.T.reshape(-1).astype(jnp.int32),
                           v_noise.T.reshape(-1).astype(jnp.int32),
                           ids])
    m = idx.shape[0]
    idx2 = idx.reshape(m // CH, CH)

    edge_w2 = jnp.concatenate([edge_w, edge_w]).reshape(1, 2 * d)

    g_in = _sc_gather(in_embed, idx2, "gather_in")
    g_out = _sc_gather(out_embed, idx2, "gather_out")
    res_in = _tc_compute(g_in.reshape(m // 2, 2 * d), edge_w2,
                         b, w_win, ns, d, "loss_in")
    res_out = _tc_compute(g_out.reshape(m // 2, 2 * d), edge_w2,
                          b, w_win, ns, d, "loss_out")
    return res_in[0, 0] + res_out[0, 0]
